# baseline (speedup 1.0000x reference)
import functools

import jax
import jax.numpy as jnp
from jax import lax
from jax.experimental import pallas as pl
from jax.experimental.pallas import tpu as pltpu
from jax.experimental.pallas import tpu_sc as plsc

_D = 128
_NW = 32
_CHUNK = 64
_K = 5


def _gather_body(n_groups, table_hbm, idx_hbm, out_hbm, idx_v, rows_v,
                 sem_in, sem_out):
    csize = _K * _CHUNK
    b_per_w = n_groups * csize
    wid = lax.axis_index("s") * 2 + lax.axis_index("c")
    base = wid * b_per_w
    pltpu.sync_copy(idx_hbm.at[pl.ds(base, b_per_w)], idx_v)

    def fire_gathers(row0, bank):
        for b in range(_K):
            pltpu.async_copy(
                table_hbm.at[idx_v.at[pl.ds(row0 + b * _CHUNK, _CHUNK)]],
                rows_v.at[bank, b], sem_in,
            )

    def drain_gathers(bank):
        for b in range(_K):
            pltpu.make_async_copy(
                table_hbm.at[idx_v.at[pl.ds(0, _CHUNK)]],
                rows_v.at[bank, b], sem_in,
            ).wait()

    def fire_outs(row0, bank):
        for b in range(_K):
            pltpu.async_copy(
                rows_v.at[bank, b],
                out_hbm.at[pl.ds(base + row0 + b * _CHUNK, _CHUNK)],
                sem_out,
            )

    def drain_outs(bank):
        for b in range(_K):
            pltpu.make_async_copy(
                rows_v.at[bank, b], out_hbm.at[pl.ds(base, _CHUNK)], sem_out
            ).wait()

    fire_gathers(0, 0)

    def iter2(i, carry):
        for sub in range(2):
            g = 2 * i + sub
            row0 = g * csize
            drain_gathers(sub)
            fire_outs(row0, sub)

            @pl.when(g >= 1)
            def _():
                drain_outs(1 - sub)

            @pl.when(g + 1 < n_groups)
            def _():
                fire_gathers(row0 + csize, 1 - sub)
        return carry

    lax.fori_loop(0, n_groups // 2, iter2, 0)
    drain_outs(1)


@functools.partial(jax.jit, static_argnames=("n_rows",))
def _gather(idx_flat, pattern, n_rows):
    b_per_w = n_rows // _NW
    n_groups = b_per_w // (_K * _CHUNK)
    run = pl.kernel(
        functools.partial(_gather_body, n_groups),
        out_type=jax.ShapeDtypeStruct((n_rows, _D), jnp.float32),
        mesh=plsc.VectorSubcoreMesh(core_axis_name="c", subcore_axis_name="s"),
        scratch_types=[
            pltpu.VMEM((b_per_w,), jnp.int32),
            pltpu.VMEM((2, _K, _CHUNK, _D), jnp.float32),
            pltpu.SemaphoreType.DMA,
            pltpu.SemaphoreType.DMA,
        ],
    )
    return run(pattern, idx_flat)


def kernel(input, pattern):
    b, t = input.shape
    idx_flat = input.reshape(-1).astype(jnp.int32)
    out = _gather(idx_flat, pattern, b * t)
    return out.reshape(b, t, _D)


# speedup vs baseline: 1.7070x; 1.7070x over previous
import functools

import jax
import jax.numpy as jnp
from jax import lax
from jax.experimental import pallas as pl
from jax.experimental.pallas import tpu as pltpu
from jax.experimental.pallas import tpu_sc as plsc

_D = 128
_NW = 32
_K = 8


def _gather_body(bpw, t, table_hbm, idx_hbm, out_hbm, idx_v, rows_v,
                 sem_in, sem_out):
    wid = lax.axis_index("s") * 2 + lax.axis_index("c")
    b0 = wid * bpw
    pltpu.sync_copy(idx_hbm.at[pl.ds(b0, bpw)], idx_v)

    n_groups = bpw // _K

    def group(g, carry):
        j0 = g * _K

        @pl.when(g > 0)
        def _():
            for k in range(_K):
                pltpu.make_async_copy(
                    rows_v.at[k], out_hbm.at[b0], sem_out
                ).wait()

        for k in range(_K):
            pltpu.async_copy(
                table_hbm.at[idx_v.at[j0 + k]], rows_v.at[k], sem_in
            )
        for k in range(_K):
            pltpu.make_async_copy(
                table_hbm.at[idx_v.at[0]], rows_v.at[k], sem_in
            ).wait()
        for k in range(_K):
            pltpu.async_copy(
                rows_v.at[k], out_hbm.at[b0 + j0 + k], sem_out
            )
        return carry

    lax.fori_loop(0, n_groups, group, 0)
    for k in range(_K):
        pltpu.make_async_copy(rows_v.at[k], out_hbm.at[b0], sem_out).wait()


@functools.partial(jax.jit, static_argnames=("b", "t"))
def _gather(idx, pattern, b, t):
    bpw = b // _NW
    run = pl.kernel(
        functools.partial(_gather_body, bpw, t),
        out_type=jax.ShapeDtypeStruct((b, t, _D), jnp.float32),
        mesh=plsc.VectorSubcoreMesh(core_axis_name="c", subcore_axis_name="s"),
        scratch_types=[
            pltpu.VMEM((bpw, t), jnp.int32),
            pltpu.VMEM((_K, t, _D), jnp.float32),
            pltpu.SemaphoreType.DMA,
            pltpu.SemaphoreType.DMA,
        ],
        compiler_params=pltpu.CompilerParams(use_tc_tiling_on_sc=True),
    )
    return run(pattern, idx)


def kernel(input, pattern):
    b, t = input.shape
    return _gather(input.astype(jnp.int32), pattern, b, t)


# speedup vs baseline: 2.9862x; 1.7494x over previous
import functools

import jax
import jax.numpy as jnp
from jax import lax
from jax.experimental import pallas as pl
from jax.experimental.pallas import tpu as pltpu
from jax.experimental.pallas import tpu_sc as plsc

_D = 128
_NW = 32
_BW = 128
_K = 5


def _gather_body(t_steps, table_hbm, idxt_hbm, out_hbm, idx_v, rows_v,
                 sem_in, sem_out):
    wid = lax.axis_index("s") * 2 + lax.axis_index("c")
    b0 = wid * _BW
    pltpu.sync_copy(idxt_hbm.at[:, pl.ds(b0, _BW)], idx_v)

    n_groups = t_steps // _K

    def group(g, carry):
        t0 = g * _K

        @pl.when(g > 0)
        def _():
            for k in range(_K):
                pltpu.make_async_copy(
                    rows_v.at[k], out_hbm.at[0, pl.ds(b0, _BW), :], sem_out
                ).wait()

        for k in range(_K):
            pltpu.async_copy(
                table_hbm.at[idx_v.at[t0 + k]], rows_v.at[k], sem_in
            )
        for k in range(_K):
            pltpu.make_async_copy(
                table_hbm.at[idx_v.at[0]], rows_v.at[k], sem_in
            ).wait()
        for k in range(_K):
            pltpu.async_copy(
                rows_v.at[k], out_hbm.at[t0 + k, pl.ds(b0, _BW), :], sem_out
            )
        return carry

    lax.fori_loop(0, n_groups, group, 0)
    for k in range(_K):
        pltpu.make_async_copy(
            rows_v.at[k], out_hbm.at[0, pl.ds(b0, _BW), :], sem_out
        ).wait()


@functools.partial(jax.jit, static_argnames=("b", "t"))
def _gather(idx_t, pattern, b, t):
    run = pl.kernel(
        functools.partial(_gather_body, t),
        out_type=jax.ShapeDtypeStruct((t, b, _D), jnp.float32),
        mesh=plsc.VectorSubcoreMesh(core_axis_name="c", subcore_axis_name="s"),
        scratch_types=[
            pltpu.VMEM((t, _BW), jnp.int32),
            pltpu.VMEM((_K, _BW, _D), jnp.float32),
            pltpu.SemaphoreType.DMA,
            pltpu.SemaphoreType.DMA,
        ],
        compiler_params=pltpu.CompilerParams(use_tc_tiling_on_sc=True),
    )
    return run(pattern, idx_t)


def kernel(input, pattern):
    b, t = input.shape
    idx_t = jnp.swapaxes(input.astype(jnp.int32), 0, 1)
    out_t = _gather(idx_t, pattern, b, t)
    return jnp.swapaxes(out_t, 0, 1)


# speedup vs baseline: 3.0518x; 1.0220x over previous
import functools

import jax
import jax.numpy as jnp
from jax import lax
from jax.experimental import pallas as pl
from jax.experimental.pallas import tpu as pltpu
from jax.experimental.pallas import tpu_sc as plsc

_D = 128
_NW = 32
_BW = 128
_HW = 64
_K = 5


def _gather_body(n_steps, table_hbm, idxt_hbm, out_hbm, idx_v, rows_v,
                 sem_in, sem_o0, sem_o1):
    wid = lax.axis_index("s") * 2 + lax.axis_index("c")
    b0 = wid * _BW
    pltpu.sync_copy(idxt_hbm.at[:, pl.ds(b0, _BW)], idx_v)

    sem_out = [sem_o0, sem_o1]
    n_groups = n_steps // _K

    def _addrs(s):
        return s // 2, (s % 2) * _HW

    def fire_gathers(g, bank):
        for k in range(_K):
            t, h = _addrs(g * _K + k)
            pltpu.async_copy(
                table_hbm.at[idx_v.at[t, pl.ds(h, _HW)]],
                rows_v.at[bank, k], sem_in,
            )

    def drain_gathers(bank):
        for k in range(_K):
            pltpu.make_async_copy(
                table_hbm.at[idx_v.at[0, pl.ds(0, _HW)]],
                rows_v.at[bank, k], sem_in,
            ).wait()

    def fire_outs(g, bank):
        for k in range(_K):
            t, h = _addrs(g * _K + k)
            pltpu.async_copy(
                rows_v.at[bank, k],
                out_hbm.at[t, pl.ds(b0 + h, _HW), :], sem_out[bank],
            )

    def drain_outs(bank):
        for k in range(_K):
            pltpu.make_async_copy(
                rows_v.at[bank, k], out_hbm.at[0, pl.ds(b0, _HW), :],
                sem_out[bank],
            ).wait()

    fire_gathers(0, 0)

    def iter2(i, carry):
        for sub in range(2):
            g = 2 * i + sub
            drain_gathers(sub)
            fire_outs(g, sub)

            @pl.when(g >= 1)
            def _():
                drain_outs(1 - sub)

            @pl.when(g + 1 < n_groups)
            def _():
                fire_gathers(g + 1, 1 - sub)
        return carry

    lax.fori_loop(0, n_groups // 2, iter2, 0)
    drain_outs(1)


@functools.partial(jax.jit, static_argnames=("b", "t"))
def _gather(idx_t, pattern, b, t):
    run = pl.kernel(
        functools.partial(_gather_body, 2 * t),
        out_type=jax.ShapeDtypeStruct((t, b, _D), jnp.float32),
        mesh=plsc.VectorSubcoreMesh(core_axis_name="c", subcore_axis_name="s"),
        scratch_types=[
            pltpu.VMEM((t, _BW), jnp.int32),
            pltpu.VMEM((2, _K, _HW, _D), jnp.float32),
            pltpu.SemaphoreType.DMA,
            pltpu.SemaphoreType.DMA,
            pltpu.SemaphoreType.DMA,
        ],
        compiler_params=pltpu.CompilerParams(use_tc_tiling_on_sc=True),
    )
    return run(pattern, idx_t)


def kernel(input, pattern):
    b, t = input.shape
    idx_t = jnp.swapaxes(input.astype(jnp.int32), 0, 1)
    out_t = _gather(idx_t, pattern, b, t)
    return jnp.swapaxes(out_t, 0, 1)
